# speedup vs baseline: 2.3585x; 2.3585x over previous
import functools

import numpy as np
import jax
import jax.numpy as jnp
from jax.experimental import pallas as pl
from jax.experimental.pallas import tpu as pltpu

_BH = 2
_WDT = jnp.bfloat16


def _taps(H, W):
    ts = []
    for dy in range(3):
        for dx in range(3):
            oy, ox = dy - 1, dx - 1
            if (H == 1 and oy != 0) or (W == 1 and ox != 0):
                continue
            ts.append((oy, ox))
    return ts


def _interp1d(n_in, n_out):
    M = np.zeros((n_out, n_in), np.float64)
    if n_in == 1:
        M[:, 0] = 1.0
        return M
    scale = (n_in - 1) / (n_out - 1)
    for i in range(n_out):
        src = i * scale
        lo = min(int(np.floor(src)), n_in - 1)
        hi = min(lo + 1, n_in - 1)
        M[i, lo] += 1.0 - (src - lo)
        M[i, hi] += src - lo
    return M


def _upmat(Bh, H, W):
    U = np.kron(_interp1d(H, 2 * H), _interp1d(W, 2 * W))
    return jnp.asarray(np.kron(np.eye(Bh), U), jnp.float32).astype(_WDT)


def _poolmat(Bh, H, W):
    Ho, Wo = H // 2, W // 2
    S = np.zeros((Bh * Ho * Wo, Bh * H * W), np.float32)
    q = np.arange(Bh * Ho * Wo)
    b = q // (Ho * Wo)
    r = q % (Ho * Wo)
    S[q, b * H * W + 2 * (r // Wo) * W + 2 * (r % Wo)] = 1.0
    return jnp.asarray(S).astype(_WDT)


def _border_mask_dict(Bh, H, W):
    out = {}
    hh, ww = np.meshgrid(np.arange(H), np.arange(W), indexing="ij")
    for oy, ox in _taps(H, W):
        if (oy, ox) == (0, 0):
            continue
        valid = ((hh + oy >= 0) & (hh + oy < H)
                 & (ww + ox >= 0) & (ww + ox < W)).astype(np.float32)
        m = np.tile(valid.reshape(1, H * W), (Bh, 1)).reshape(Bh * H * W, 1)
        out[(oy, ox)] = jnp.asarray(m).astype(_WDT)
    return out


def _shift(x, s):
    P = x.shape[0]
    s = s % P
    if s == 0:
        return x
    return jnp.concatenate([x[s:], x[:s]], axis=0)


def _conv(xs, wrefs, sref, bref, H, W, mdict):
    taps = _taps(H, W)
    acc = None
    for xv, wr in zip(xs, wrefs):
        if len(taps) == 1:
            xcat = xv
        else:
            cols = []
            for oy, ox in taps:
                xsft = _shift(xv, oy * W + ox)
                if (oy, ox) != (0, 0):
                    xsft = xsft * mdict[(oy, ox)][...]
                cols.append(xsft)
            xcat = jnp.concatenate(cols, axis=1)
        d = jnp.dot(xcat, wr[...], preferred_element_type=jnp.float32)
        acc = d if acc is None else acc + d
    y = acc * sref[...] + bref[...]
    return jnp.maximum(y, 0.0).astype(_WDT)


def _pool(x, selref, W):
    m = jnp.maximum(x, _shift(x, 1))
    m = jnp.maximum(m, _shift(m, W))
    return jnp.dot(selref[...], m,
                   preferred_element_type=jnp.float32).astype(_WDT)


def _unet_body(*refs, treedef, hw, L):
    x_ref, out_ref = refs[0], refs[-1]
    p = jax.tree_util.tree_unflatten(treedef, refs[1:-1])

    def dconv(xs, cp, lvl):
        w1s, s1, b1, w2, s2, b2 = cp
        H, W = hw[lvl]
        m = p["mask"][lvl]
        h = _conv(list(xs), list(w1s), s1, b1, H, W, m)
        return _conv([h], [w2], s2, b2, H, W, m)

    cur = dconv([x_ref[...].astype(_WDT)], p["inc"], 0)
    skips = [cur]
    for i in range(L):
        pooled = _pool(cur, p["pool"][i], hw[i][1])
        cur = dconv([pooled], p["downs"][i], i + 1)
        skips.append(cur)

    for i in range(L):
        up = jnp.dot(p["upmat"][i][...], cur,
                     preferred_element_type=jnp.float32).astype(_WDT)
        cur = dconv([skips[L - 1 - i], up], p["ups"][i], L - 1 - i)

    wo, bo = p["outc"]
    out_ref[...] = jnp.dot(cur, wo[...],
                           preferred_element_type=jnp.float32) + bo[...]


def _cast_w(w, H, W):
    taps = _taps(H, W)
    if len(taps) == 1:
        return w[4].astype(_WDT)
    cin, cout = w.shape[1], w.shape[2]
    return w.reshape(9 * cin, cout).astype(_WDT)


def kernel(inp, p0, p1, p2, p3, p4, p5, p6, p7, p8, p9, p10, p11, p12, p13,
           p14, p15, p16, p17, p18, p19, p20, p21, p22, p23, p24, p25, p26,
           p27, p28, p29, p30, p31, p32, p33, p34, p35, p36, p37, p38, p39,
           p40, p41, p42, p43, p44, p45, p46, p47, p48, p49, p50, p51, p52,
           p53, p54, p55, p56, p57, p58, p59):
    p = [p0, p1, p2, p3, p4, p5, p6, p7, p8, p9, p10, p11, p12, p13, p14,
         p15, p16, p17, p18, p19, p20, p21, p22, p23, p24, p25, p26, p27,
         p28, p29, p30, p31, p32, p33, p34, p35, p36, p37, p38, p39, p40,
         p41, p42, p43, p44, p45, p46, p47, p48, p49, p50, p51, p52, p53,
         p54, p55, p56, p57, p58, p59]

    B, D1, D2 = inp.shape
    H, W = D2, D1
    L = 4
    hw = [(H >> i, W >> i) for i in range(L + 1)]

    def cw(idx, lvl):
        return _cast_w(p[idx], *hw[lvl])

    downs = tuple(
        ((cw(6 * i, i + 1),), p[6 * i + 1], p[6 * i + 2],
         cw(6 * i + 3, i + 1), p[6 * i + 4], p[6 * i + 5])
        for i in range(L))
    inc = ((cw(24, 0),), p[25], p[26], cw(27, 0), p[28], p[29])
    outc = (p[30].reshape(-1, 1).astype(_WDT), p[31])
    ups = tuple(
        ((cw(32 + 7 * i, L - 1 - i), cw(33 + 7 * i, L - 1 - i)),
         p[34 + 7 * i], p[35 + 7 * i],
         cw(36 + 7 * i, L - 1 - i), p[37 + 7 * i], p[38 + 7 * i])
        for i in range(L))

    kp = {
        "inc": inc, "downs": downs, "ups": ups, "outc": outc,
        "pool": tuple(_poolmat(_BH, *hw[i]) for i in range(L)),
        "upmat": tuple(_upmat(_BH, *hw[L - i]) for i in range(L)),
        "mask": tuple(_border_mask_dict(_BH, *hw[i]) for i in range(L + 1)),
    }

    x_pix = jnp.transpose(inp, (0, 2, 1)).reshape(B * H * W, 1)
    flat, treedef = jax.tree_util.tree_flatten(kp)

    def _full_spec(a):
        nd = a.ndim
        return pl.BlockSpec(a.shape, lambda i, _n=nd: (0,) * _n)

    Pb = _BH * H * W
    body = functools.partial(_unet_body, treedef=treedef, hw=hw, L=L)
    y = pl.pallas_call(
        body,
        out_shape=jax.ShapeDtypeStruct((B * H * W, 1), jnp.float32),
        grid=(B // _BH,),
        in_specs=[pl.BlockSpec((Pb, 1), lambda i: (i, 0))]
        + [_full_spec(a) for a in flat],
        out_specs=pl.BlockSpec((Pb, 1), lambda i: (i, 0)),
        compiler_params=pltpu.CompilerParams(
            dimension_semantics=("parallel",),
            vmem_limit_bytes=64 * 1024 * 1024,
        ),
    )(x_pix, *flat)

    return jnp.transpose(y.reshape(B, H, W), (0, 2, 1))


# speedup vs baseline: 3.7076x; 1.5720x over previous
import functools

import numpy as np
import jax
import jax.numpy as jnp
from jax.experimental import pallas as pl
from jax.experimental.pallas import tpu as pltpu

_BH = 2
_WDT = jnp.bfloat16


def _taps(H, W):
    ts = []
    for dy in range(3):
        for dx in range(3):
            oy, ox = dy - 1, dx - 1
            if (H == 1 and oy != 0) or (W == 1 and ox != 0):
                continue
            ts.append((oy, ox))
    return ts


def _interp1d(n_in, n_out):
    M = np.zeros((n_out, n_in), np.float64)
    if n_in == 1:
        M[:, 0] = 1.0
        return M
    scale = (n_in - 1) / (n_out - 1)
    for i in range(n_out):
        src = i * scale
        lo = min(int(np.floor(src)), n_in - 1)
        hi = min(lo + 1, n_in - 1)
        M[i, lo] += 1.0 - (src - lo)
        M[i, hi] += src - lo
    return M


def _upmat(Bh, H, W):
    U = np.kron(_interp1d(H, 2 * H), _interp1d(W, 2 * W))
    return jnp.asarray(np.kron(np.eye(Bh), U), jnp.float32).astype(_WDT)


def _poolmat(Bh, H, W):
    Ho, Wo = H // 2, W // 2
    S = np.zeros((Bh * Ho * Wo, Bh * H * W), np.float32)
    q = np.arange(Bh * Ho * Wo)
    b = q // (Ho * Wo)
    r = q % (Ho * Wo)
    S[q, b * H * W + 2 * (r // Wo) * W + 2 * (r % Wo)] = 1.0
    return jnp.asarray(S).astype(_WDT)


def _border_mask_dict(Bh, H, W):
    out = {}
    hh, ww = np.meshgrid(np.arange(H), np.arange(W), indexing="ij")
    for oy, ox in _taps(H, W):
        if (oy, ox) == (0, 0):
            continue
        valid = ((hh + oy >= 0) & (hh + oy < H)
                 & (ww + ox >= 0) & (ww + ox < W)).astype(np.float32)
        m = np.tile(valid.reshape(1, H * W), (Bh, 1)).reshape(Bh * H * W, 1)
        out[(oy, ox)] = jnp.asarray(m).astype(_WDT)
    return out


def _shift(x, s):
    P = x.shape[0]
    s = s % P
    if s == 0:
        return x
    return jnp.concatenate([x[s:], x[:s]], axis=0)


def _conv(xs, wrefs, sref, bref, H, W, mdict):
    taps = _taps(H, W)
    acc = None
    for xv, wr in zip(xs, wrefs):
        if len(taps) == 1:
            xcat = xv
        else:
            cols = []
            for oy, ox in taps:
                xsft = _shift(xv, oy * W + ox)
                if (oy, ox) != (0, 0):
                    xsft = xsft * mdict[(oy, ox)][...]
                cols.append(xsft)
            xcat = jnp.concatenate(cols, axis=1)
        d = jnp.dot(xcat, wr[...].astype(_WDT),
                    preferred_element_type=jnp.float32)
        acc = d if acc is None else acc + d
    y = acc * sref[...] + bref[...]
    return jnp.maximum(y, 0.0).astype(_WDT)


def _pool(x, selref, W):
    m = jnp.maximum(x, _shift(x, 1))
    m = jnp.maximum(m, _shift(m, W))
    return jnp.dot(selref[...], m,
                   preferred_element_type=jnp.float32).astype(_WDT)


def _unet_body(*refs, treedef, hw, L):
    x_ref, out_ref = refs[0], refs[-1]
    p = jax.tree_util.tree_unflatten(treedef, refs[1:-1])

    def dconv(xs, cp, lvl):
        w1s, s1, b1, w2, s2, b2 = cp
        H, W = hw[lvl]
        m = p["mask"][lvl]
        h = _conv(list(xs), list(w1s), s1, b1, H, W, m)
        return _conv([h], [w2], s2, b2, H, W, m)

    cur = dconv([x_ref[...].astype(_WDT)], p["inc"], 0)
    skips = [cur]
    for i in range(L):
        pooled = _pool(cur, p["pool"][i], hw[i][1])
        cur = dconv([pooled], p["downs"][i], i + 1)
        skips.append(cur)

    for i in range(L):
        up = jnp.dot(p["upmat"][i][...], cur,
                     preferred_element_type=jnp.float32).astype(_WDT)
        cur = dconv([skips[L - 1 - i], up], p["ups"][i], L - 1 - i)

    wo, bo = p["outc"]
    out_ref[...] = jnp.dot(cur, wo[...].astype(_WDT),
                           preferred_element_type=jnp.float32) + bo[...]


def _cast_w(w, H, W):
    taps = _taps(H, W)
    if len(taps) == 1:
        return w[4]
    cin, cout = w.shape[1], w.shape[2]
    return w.reshape(9 * cin, cout)


def kernel(inp, p0, p1, p2, p3, p4, p5, p6, p7, p8, p9, p10, p11, p12, p13,
           p14, p15, p16, p17, p18, p19, p20, p21, p22, p23, p24, p25, p26,
           p27, p28, p29, p30, p31, p32, p33, p34, p35, p36, p37, p38, p39,
           p40, p41, p42, p43, p44, p45, p46, p47, p48, p49, p50, p51, p52,
           p53, p54, p55, p56, p57, p58, p59):
    p = [p0, p1, p2, p3, p4, p5, p6, p7, p8, p9, p10, p11, p12, p13, p14,
         p15, p16, p17, p18, p19, p20, p21, p22, p23, p24, p25, p26, p27,
         p28, p29, p30, p31, p32, p33, p34, p35, p36, p37, p38, p39, p40,
         p41, p42, p43, p44, p45, p46, p47, p48, p49, p50, p51, p52, p53,
         p54, p55, p56, p57, p58, p59]

    B, D1, D2 = inp.shape
    H, W = D2, D1
    L = 4
    hw = [(H >> i, W >> i) for i in range(L + 1)]

    def cw(idx, lvl):
        return _cast_w(p[idx], *hw[lvl])

    downs = tuple(
        ((cw(6 * i, i + 1),), p[6 * i + 1], p[6 * i + 2],
         cw(6 * i + 3, i + 1), p[6 * i + 4], p[6 * i + 5])
        for i in range(L))
    inc = ((cw(24, 0),), p[25], p[26], cw(27, 0), p[28], p[29])
    outc = (p[30].reshape(-1, 1), p[31])
    ups = tuple(
        ((cw(32 + 7 * i, L - 1 - i), cw(33 + 7 * i, L - 1 - i)),
         p[34 + 7 * i], p[35 + 7 * i],
         cw(36 + 7 * i, L - 1 - i), p[37 + 7 * i], p[38 + 7 * i])
        for i in range(L))

    kp = {
        "inc": inc, "downs": downs, "ups": ups, "outc": outc,
        "pool": tuple(_poolmat(_BH, *hw[i]) for i in range(L)),
        "upmat": tuple(_upmat(_BH, *hw[L - i]) for i in range(L)),
        "mask": tuple(_border_mask_dict(_BH, *hw[i]) for i in range(L + 1)),
    }

    x_pix = jnp.transpose(inp, (0, 2, 1)).reshape(B * H * W, 1)
    flat, treedef = jax.tree_util.tree_flatten(kp)

    def _full_spec(a):
        nd = a.ndim
        return pl.BlockSpec(a.shape, lambda i, _n=nd: (0,) * _n)

    Pb = _BH * H * W
    body = functools.partial(_unet_body, treedef=treedef, hw=hw, L=L)
    y = pl.pallas_call(
        body,
        out_shape=jax.ShapeDtypeStruct((B * H * W, 1), jnp.float32),
        grid=(B // _BH,),
        in_specs=[pl.BlockSpec((Pb, 1), lambda i: (i, 0))]
        + [_full_spec(a) for a in flat],
        out_specs=pl.BlockSpec((Pb, 1), lambda i: (i, 0)),
        compiler_params=pltpu.CompilerParams(
            dimension_semantics=("parallel",),
            vmem_limit_bytes=64 * 1024 * 1024,
        ),
    )(x_pix, *flat)

    return jnp.transpose(y.reshape(B, H, W), (0, 2, 1))


# speedup vs baseline: 4.0336x; 1.0879x over previous
import functools

import numpy as np
import jax
import jax.numpy as jnp
from jax import lax
from jax.experimental import pallas as pl
from jax.experimental.pallas import tpu as pltpu

_BH = 2
_WDT = jnp.bfloat16


def _taps(H, W):
    ts = []
    for dy in range(3):
        for dx in range(3):
            oy, ox = dy - 1, dx - 1
            if (H == 1 and oy != 0) or (W == 1 and ox != 0):
                continue
            ts.append((oy, ox))
    return ts


def _interp1d(n_in, n_out):
    M = np.zeros((n_out, n_in), np.float64)
    if n_in == 1:
        M[:, 0] = 1.0
        return M
    scale = (n_in - 1) / (n_out - 1)
    for i in range(n_out):
        src = i * scale
        lo = min(int(np.floor(src)), n_in - 1)
        hi = min(lo + 1, n_in - 1)
        M[i, lo] += 1.0 - (src - lo)
        M[i, hi] += src - lo
    return M


def _upmat(Bh, H, W):
    U = np.kron(_interp1d(H, 2 * H), _interp1d(W, 2 * W))
    return jnp.asarray(np.kron(np.eye(Bh), U), jnp.float32).astype(_WDT)


def _poolmat(Bh, H, W):
    Ho, Wo = H // 2, W // 2
    S = np.zeros((Bh * Ho * Wo, Bh * H * W), np.float32)
    q = np.arange(Bh * Ho * Wo)
    b = q // (Ho * Wo)
    r = q % (Ho * Wo)
    S[q, b * H * W + 2 * (r // Wo) * W + 2 * (r % Wo)] = 1.0
    return jnp.asarray(S).astype(_WDT)


def _shift(x, s):
    P = x.shape[0]
    s = s % P
    if s == 0:
        return x
    return jnp.concatenate([x[s:], x[:s]], axis=0)


def _mk_masks(Bh, H, W):
    P = Bh * H * W
    if H == 1 and W == 1:
        return {}
    p = lax.broadcasted_iota(jnp.int32, (P, 1), 0)
    h = (p // W) % H
    w = p % W
    one = jnp.full((P, 1), 1.0, jnp.float32)
    zero = jnp.zeros((P, 1), jnp.float32)

    def cond1(v, o, n):
        if o == 0:
            return None
        c = (v >= 1) if o < 0 else (v <= n - 2)
        return jnp.where(c, one, zero).astype(_WDT)

    out = {}
    for oy, ox in _taps(H, W):
        if (oy, ox) == (0, 0):
            continue
        mh, mw = cond1(h, oy, H), cond1(w, ox, W)
        m = mh if mw is None else (mw if mh is None else mh * mw)
        out[(oy, ox)] = m
    return out


def _conv(xs_w, sref, bref, H, W, masks):
    taps = _taps(H, W)
    acc = None
    for xv, wr in xs_w:
        for oy, ox in taps:
            ti = 0 if wr.shape[0] == 1 else 3 * (oy + 1) + (ox + 1)
            wt = wr[ti].astype(_WDT)
            xsft = _shift(xv, oy * W + ox)
            if (oy, ox) != (0, 0):
                xsft = xsft * masks[(oy, ox)]
            d = jnp.dot(xsft, wt, preferred_element_type=jnp.float32)
            acc = d if acc is None else acc + d
    y = acc * sref[...] + bref[...]
    return jnp.maximum(y, 0.0).astype(_WDT)


def _pool(x, selref, W):
    m = jnp.maximum(x, _shift(x, 1))
    m = jnp.maximum(m, _shift(m, W))
    return jnp.dot(selref[...], m,
                   preferred_element_type=jnp.float32).astype(_WDT)


def _unet_body(*refs, treedef, hw, L):
    x_ref, out_ref = refs[0], refs[-1]
    p = jax.tree_util.tree_unflatten(treedef, refs[1:-1])
    masks = [_mk_masks(_BH, *hw[l]) for l in range(L + 1)]

    def dconv(xs_w1, cp, lvl):
        _, s1, b1, w2, s2, b2 = cp
        H, W = hw[lvl]
        h = _conv(xs_w1, s1, b1, H, W, masks[lvl])
        return _conv([(h, w2)], s2, b2, H, W, masks[lvl])

    def block_w1(cp):
        return cp[0]

    cur = dconv([(x_ref[...].astype(_WDT), block_w1(p["inc"])[0])],
                p["inc"], 0)
    skips = [cur]
    for i in range(L):
        pooled = _pool(cur, p["pool"][i], hw[i][1])
        cur = dconv([(pooled, block_w1(p["downs"][i])[0])],
                    p["downs"][i], i + 1)
        skips.append(cur)

    for i in range(L):
        up = jnp.dot(p["upmat"][i][...], cur,
                     preferred_element_type=jnp.float32).astype(_WDT)
        w1a, w1b = block_w1(p["ups"][i])
        cur = dconv([(skips[L - 1 - i], w1a), (up, w1b)],
                    p["ups"][i], L - 1 - i)

    wo, bo = p["outc"]
    y = jnp.sum(cur.astype(jnp.float32) * wo[...], axis=1, keepdims=True)
    out_ref[...] = y + bo[...]


def kernel(inp, p0, p1, p2, p3, p4, p5, p6, p7, p8, p9, p10, p11, p12, p13,
           p14, p15, p16, p17, p18, p19, p20, p21, p22, p23, p24, p25, p26,
           p27, p28, p29, p30, p31, p32, p33, p34, p35, p36, p37, p38, p39,
           p40, p41, p42, p43, p44, p45, p46, p47, p48, p49, p50, p51, p52,
           p53, p54, p55, p56, p57, p58, p59):
    p = [p0, p1, p2, p3, p4, p5, p6, p7, p8, p9, p10, p11, p12, p13, p14,
         p15, p16, p17, p18, p19, p20, p21, p22, p23, p24, p25, p26, p27,
         p28, p29, p30, p31, p32, p33, p34, p35, p36, p37, p38, p39, p40,
         p41, p42, p43, p44, p45, p46, p47, p48, p49, p50, p51, p52, p53,
         p54, p55, p56, p57, p58, p59]

    B, D1, D2 = inp.shape
    H, W = D2, D1
    L = 4
    hw = [(H >> i, W >> i) for i in range(L + 1)]

    downs = tuple(
        ((p[6 * i],), p[6 * i + 1], p[6 * i + 2],
         p[6 * i + 3], p[6 * i + 4], p[6 * i + 5])
        for i in range(L))
    inc = ((p[24],), p[25], p[26], p[27], p[28], p[29])
    outc = (p[30], p[31])
    ups = tuple(
        ((p[32 + 7 * i], p[33 + 7 * i]), p[34 + 7 * i], p[35 + 7 * i],
         p[36 + 7 * i], p[37 + 7 * i], p[38 + 7 * i])
        for i in range(L))

    kp = {
        "inc": inc, "downs": downs, "ups": ups, "outc": outc,
        "pool": tuple(_poolmat(_BH, *hw[i]) for i in range(L)),
        "upmat": tuple(_upmat(_BH, *hw[L - i]) for i in range(L)),
    }

    x_pix = jnp.transpose(inp, (0, 2, 1)).reshape(B * H * W, 1)
    flat, treedef = jax.tree_util.tree_flatten(kp)

    tap4 = {id(p[18]), id(p[21])}

    def _spec(a):
        nd = a.ndim
        if id(a) in tap4:
            return pl.BlockSpec((1,) + a.shape[1:], lambda i: (4, 0, 0))
        return pl.BlockSpec(a.shape, lambda i, _n=nd: (0,) * _n)

    Pb = _BH * H * W
    body = functools.partial(_unet_body, treedef=treedef, hw=hw, L=L)
    y = pl.pallas_call(
        body,
        out_shape=jax.ShapeDtypeStruct((B * H * W, 1), jnp.float32),
        grid=(B // _BH,),
        in_specs=[pl.BlockSpec((Pb, 1), lambda i: (i, 0))]
        + [_spec(a) for a in flat],
        out_specs=pl.BlockSpec((Pb, 1), lambda i: (i, 0)),
        compiler_params=pltpu.CompilerParams(
            dimension_semantics=("parallel",),
            vmem_limit_bytes=64 * 1024 * 1024,
        ),
    )(x_pix, *flat)

    return jnp.transpose(y.reshape(B, H, W), (0, 2, 1))


# speedup vs baseline: 4.6732x; 1.1586x over previous
import functools

import numpy as np
import jax
import jax.numpy as jnp
from jax import lax
from jax.experimental import pallas as pl
from jax.experimental.pallas import tpu as pltpu

_BH = 4
_WDT = jnp.bfloat16


def _taps(H, W):
    ts = []
    for dy in range(3):
        for dx in range(3):
            oy, ox = dy - 1, dx - 1
            if (H == 1 and oy != 0) or (W == 1 and ox != 0):
                continue
            ts.append((oy, ox))
    return ts


def _interp1d(n_in, n_out):
    M = np.zeros((n_out, n_in), np.float64)
    if n_in == 1:
        M[:, 0] = 1.0
        return M
    scale = (n_in - 1) / (n_out - 1)
    for i in range(n_out):
        src = i * scale
        lo = min(int(np.floor(src)), n_in - 1)
        hi = min(lo + 1, n_in - 1)
        M[i, lo] += 1.0 - (src - lo)
        M[i, hi] += src - lo
    return M


def _upmat(Bh, H, W):
    U = np.kron(_interp1d(H, 2 * H), _interp1d(W, 2 * W))
    return jnp.asarray(np.kron(np.eye(Bh), U), jnp.float32).astype(_WDT)


def _poolmat(Bh, H, W):
    Ho, Wo = H // 2, W // 2
    S = np.zeros((Bh * Ho * Wo, Bh * H * W), np.float32)
    q = np.arange(Bh * Ho * Wo)
    b = q // (Ho * Wo)
    r = q % (Ho * Wo)
    S[q, b * H * W + 2 * (r // Wo) * W + 2 * (r % Wo)] = 1.0
    return jnp.asarray(S).astype(_WDT)


def _shift(x, s):
    P = x.shape[0]
    s = s % P
    if s == 0:
        return x
    return jnp.concatenate([x[s:], x[:s]], axis=0)


def _mk_masks(Bh, H, W):
    P = Bh * H * W
    if H == 1 and W == 1:
        return {}
    p = lax.broadcasted_iota(jnp.int32, (P, 1), 0)
    h = (p // W) % H
    w = p % W
    one = jnp.full((P, 1), 1.0, jnp.float32)
    zero = jnp.zeros((P, 1), jnp.float32)

    def cond1(v, o, n):
        if o == 0:
            return None
        c = (v >= 1) if o < 0 else (v <= n - 2)
        return jnp.where(c, one, zero).astype(_WDT)

    out = {}
    for oy, ox in _taps(H, W):
        if (oy, ox) == (0, 0):
            continue
        mh, mw = cond1(h, oy, H), cond1(w, ox, W)
        m = mh if mw is None else (mw if mh is None else mh * mw)
        out[(oy, ox)] = m
    return out


def _conv(xs_w, sref, bref, H, W, masks):
    taps = _taps(H, W)
    acc = None
    for xv, wr in xs_w:
        for oy, ox in taps:
            ti = 0 if wr.shape[0] == 1 else 3 * (oy + 1) + (ox + 1)
            wt = wr[ti].astype(_WDT)
            xsft = _shift(xv, oy * W + ox)
            if (oy, ox) != (0, 0):
                xsft = xsft * masks[(oy, ox)]
            d = jnp.dot(xsft, wt, preferred_element_type=jnp.float32)
            acc = d if acc is None else acc + d
    y = acc * sref[...] + bref[...]
    return jnp.maximum(y, 0.0).astype(_WDT)


def _pool(x, selref, W):
    m = jnp.maximum(x, _shift(x, 1))
    m = jnp.maximum(m, _shift(m, W))
    return jnp.dot(selref[...], m,
                   preferred_element_type=jnp.float32).astype(_WDT)


def _unet_body(*refs, treedef, hw, L):
    x_ref, out_ref = refs[0], refs[-1]
    p = jax.tree_util.tree_unflatten(treedef, refs[1:-1])
    masks = [_mk_masks(_BH, *hw[l]) for l in range(L + 1)]

    def dconv(xs_w1, cp, lvl):
        _, s1, b1, w2, s2, b2 = cp
        H, W = hw[lvl]
        h = _conv(xs_w1, s1, b1, H, W, masks[lvl])
        return _conv([(h, w2)], s2, b2, H, W, masks[lvl])

    def block_w1(cp):
        return cp[0]

    cur = dconv([(x_ref[...].astype(_WDT), block_w1(p["inc"])[0])],
                p["inc"], 0)
    skips = [cur]
    for i in range(L):
        pooled = _pool(cur, p["pool"][i], hw[i][1])
        cur = dconv([(pooled, block_w1(p["downs"][i])[0])],
                    p["downs"][i], i + 1)
        skips.append(cur)

    for i in range(L):
        up = jnp.dot(p["upmat"][i][...], cur,
                     preferred_element_type=jnp.float32).astype(_WDT)
        w1a, w1b = block_w1(p["ups"][i])
        cur = dconv([(skips[L - 1 - i], w1a), (up, w1b)],
                    p["ups"][i], L - 1 - i)

    wo, bo = p["outc"]
    y = jnp.sum(cur.astype(jnp.float32) * wo[...], axis=1, keepdims=True)
    out_ref[...] = y + bo[...]


def kernel(inp, p0, p1, p2, p3, p4, p5, p6, p7, p8, p9, p10, p11, p12, p13,
           p14, p15, p16, p17, p18, p19, p20, p21, p22, p23, p24, p25, p26,
           p27, p28, p29, p30, p31, p32, p33, p34, p35, p36, p37, p38, p39,
           p40, p41, p42, p43, p44, p45, p46, p47, p48, p49, p50, p51, p52,
           p53, p54, p55, p56, p57, p58, p59):
    p = [p0, p1, p2, p3, p4, p5, p6, p7, p8, p9, p10, p11, p12, p13, p14,
         p15, p16, p17, p18, p19, p20, p21, p22, p23, p24, p25, p26, p27,
         p28, p29, p30, p31, p32, p33, p34, p35, p36, p37, p38, p39, p40,
         p41, p42, p43, p44, p45, p46, p47, p48, p49, p50, p51, p52, p53,
         p54, p55, p56, p57, p58, p59]

    B, D1, D2 = inp.shape
    H, W = D2, D1
    L = 4
    hw = [(H >> i, W >> i) for i in range(L + 1)]

    downs = tuple(
        ((p[6 * i],), p[6 * i + 1], p[6 * i + 2],
         p[6 * i + 3], p[6 * i + 4], p[6 * i + 5])
        for i in range(L))
    inc = ((p[24],), p[25], p[26], p[27], p[28], p[29])
    outc = (p[30], p[31])
    ups = tuple(
        ((p[32 + 7 * i], p[33 + 7 * i]), p[34 + 7 * i], p[35 + 7 * i],
         p[36 + 7 * i], p[37 + 7 * i], p[38 + 7 * i])
        for i in range(L))

    kp = {
        "inc": inc, "downs": downs, "ups": ups, "outc": outc,
        "pool": tuple(_poolmat(_BH, *hw[i]) for i in range(L)),
        "upmat": tuple(_upmat(_BH, *hw[L - i]) for i in range(L)),
    }

    x_pix = jnp.transpose(inp, (0, 2, 1)).reshape(B * H * W, 1)
    flat, treedef = jax.tree_util.tree_flatten(kp)

    tap4 = {id(p[18]), id(p[21])}

    def _spec(a):
        nd = a.ndim
        if id(a) in tap4:
            return pl.BlockSpec((1,) + a.shape[1:], lambda i: (4, 0, 0))
        return pl.BlockSpec(a.shape, lambda i, _n=nd: (0,) * _n)

    Pb = _BH * H * W
    body = functools.partial(_unet_body, treedef=treedef, hw=hw, L=L)
    y = pl.pallas_call(
        body,
        out_shape=jax.ShapeDtypeStruct((B * H * W, 1), jnp.float32),
        grid=(B // _BH,),
        in_specs=[pl.BlockSpec((Pb, 1), lambda i: (i, 0))]
        + [_spec(a) for a in flat],
        out_specs=pl.BlockSpec((Pb, 1), lambda i: (i, 0)),
        compiler_params=pltpu.CompilerParams(
            dimension_semantics=("arbitrary",),
            vmem_limit_bytes=64 * 1024 * 1024,
        ),
    )(x_pix, *flat)

    return jnp.transpose(y.reshape(B, H, W), (0, 2, 1))
